# baseline (speedup 1.0000x reference)
import functools

import jax
import jax.numpy as jnp
import numpy as np
from jax import lax
from jax.experimental import pallas as pl
from jax.experimental.pallas import tpu as pltpu
from jax.experimental.pallas import tpu_sc as plsc

_COV_BASE = [0.2, 0.31, 0.28, 1.28, 0.96, 0.84, 0.76, 0.71, 0.66, 0.57, 0.58, 1.66, 1.41, 1.21, 1.11, 1.07, 1.05, 1.02, 1.06, 2.03, 1.76, 1.70, 1.60, 1.53, 1.39, 1.39, 1.32, 1.26, 1.24, 1.32, 1.22, 1.22, 1.20, 1.19, 1.20, 1.20, 1.16, 2.20, 1.95, 1.90, 1.75, 1.64, 1.54, 1.47, 1.46, 1.42, 1.39, 1.45, 1.44, 1.42, 1.39, 1.39, 1.38, 1.39, 1.40, 2.44, 2.15, 2.07, 2.04, 2.03, 2.01, 1.99, 1.98, 1.98, 1.96, 1.94, 1.92, 1.92, 1.89, 1.90, 1.87, 1.87, 1.75, 1.70, 1.62, 1.51, 1.44, 1.41, 1.36, 1.36, 1.32, 1.45, 1.46, 1.48, 1.40, 1.50, 1.50, 2.60, 2.21, 2.15, 2.06, 2.00, 1.96, 1.90, 1.87, 1.80, 1.69]
_COVALENT_RADII = np.array(_COV_BASE + [1.69] * (119 - len(_COV_BASE)), dtype=np.float32)

_N_NODES = 100000
_N_EDGES = 6400000
_NC = 2
_NS = 16
_NW = _NC * _NS
_EPW = _N_EDGES // _NW
_CHUNK = 2000
_NCHUNK = _EPW // _CHUNK
_NPAD = 100352
_SLICE = _NPAD // _NS
_L = 16


_NBUF = 4
_UNROLL = 25

_PHI_SCALE = 128.0
_PHI_N = 4104
_PHI_C = np.array([0.1818, 0.5099, 0.2802, 0.02817], dtype=np.float64)
_PHI_B = np.array([3.2, 0.9423, 0.4028, 0.2016], dtype=np.float64)
_PHI_U = np.arange(_PHI_N) / _PHI_SCALE
_PHI_TAB = (_PHI_C[None, :] * np.exp(-_PHI_B[None, :] * _PHI_U[:, None])).sum(1).astype(np.float32)
_DPHI_TAB = np.concatenate([_PHI_TAB[1:] - _PHI_TAB[:-1], np.zeros(1, np.float32)])


def _sc_body(x_hbm, s_hbm, r_hbm, elem_hbm, inva_hbm, invr_hbm, zz_hbm,
             phi0_hbm, dphi_hbm, out_hbm, elem_v, inva_v, invr_v, zz_v,
             phi0_v, dphi_v, xbufs, sbufs, rbufs, vbufs, acc, in_sems,
             scat_sems):
    cid = lax.axis_index("c")
    sid = lax.axis_index("s")
    wid = cid * _NS + sid

    pltpu.sync_copy(elem_hbm, elem_v)
    pltpu.sync_copy(inva_hbm, inva_v)
    pltpu.sync_copy(invr_hbm, invr_v)
    pltpu.sync_copy(zz_hbm, zz_v)
    pltpu.sync_copy(phi0_hbm, phi0_v)
    pltpu.sync_copy(dphi_hbm, dphi_v)

    def _zero(i, _):
        vbufs[0][pl.ds(i * _L, _L)] = jnp.zeros((_L,), jnp.float32)
        return 0
    lax.fori_loop(0, _CHUNK // _L, _zero, 0)
    for j in range(4):
        pltpu.sync_copy(vbufs[0].at[pl.ds(0, 1568)],
                        acc.at[pl.ds(sid * _SLICE + j * 1568, 1568)])
    plsc.subcore_barrier()

    base_e = wid * _EPW

    def _issue_in(c, slot):
        off = base_e + c * _CHUNK
        pltpu.async_copy(x_hbm.at[pl.ds(off, _CHUNK)], xbufs[slot],
                         in_sems[slot])
        pltpu.async_copy(s_hbm.at[pl.ds(off, _CHUNK)], sbufs[slot],
                         in_sems[slot])
        pltpu.async_copy(r_hbm.at[pl.ds(off, _CHUNK)], rbufs[slot],
                         in_sems[slot])

    def _wait_in(slot):
        pltpu.make_async_copy(x_hbm.at[pl.ds(0, _CHUNK)], xbufs[slot],
                              in_sems[slot]).wait()
        pltpu.make_async_copy(s_hbm.at[pl.ds(0, _CHUNK)], sbufs[slot],
                              in_sems[slot]).wait()
        pltpu.make_async_copy(r_hbm.at[pl.ds(0, _CHUNK)], rbufs[slot],
                              in_sems[slot]).wait()

    def _wait_scat(slot):
        pltpu.make_async_copy(vbufs[slot % 2], acc.at[rbufs[slot]],
                              scat_sems[slot]).wait()

    for b in range(_NBUF - 1):
        _issue_in(jnp.int32(b), b)

    def _compute(slot):
        xbuf, sbuf, rbuf = xbufs[slot], sbufs[slot], rbufs[slot]
        vbuf = vbufs[slot % 2]

        @plsc.parallel_loop(0, _CHUNK // _L, unroll=_UNROLL)
        def _vreg(i):
            o = i * _L
            s = sbuf[pl.ds(o, _L)]
            r = rbuf[pl.ds(o, _L)]
            wu = plsc.load_gather(elem_v, [s >> 2])
            wv = plsc.load_gather(elem_v, [r >> 2])
            eu = (wu >> ((s & 3) << 3)) & 0xFF
            ev = (wv >> ((r & 3) << 3)) & 0xFF
            pair = eu * 10 + ev
            inva = plsc.load_gather(inva_v, [pair])
            invr = plsc.load_gather(invr_v, [pair])
            zz = plsc.load_gather(zz_v, [pair])
            xv = xbuf[pl.ds(o, _L)]
            us = xv * inva
            ui = us.astype(jnp.int32)
            frac = us - ui.astype(jnp.float32)
            p0 = plsc.load_gather(phi0_v, [ui])
            d0 = plsc.load_gather(dphi_v, [ui])
            phi = p0 + frac * d0
            t = xv * invr
            t2 = t * t
            t6 = t2 * t2 * t2
            env = 1.0 + t6 * (-28.0 + t * (48.0 - 21.0 * t))
            env = jnp.where(t < 1.0, env, 0.0)
            vbuf[pl.ds(o, _L)] = zz * phi * env / xv

    def _outer(i, _):
        for b in range(_NBUF):
            _wait_in(b)
            _compute(b)
            pltpu.async_copy(vbufs[b % 2], acc.at[rbufs[b]], scat_sems[b],
                             add=True)
            prev = (b + _NBUF - 1) % _NBUF
            if b == 0:
                @pl.when(i > 0)
                def _():
                    _wait_scat(prev)
                    _issue_in(_NBUF * i + b + _NBUF - 1, prev)
                @pl.when(i == 0)
                def _():
                    _issue_in(_NBUF * i + b + _NBUF - 1, prev)
            else:
                _wait_scat(prev)
                nxt = _NBUF * i + b + _NBUF - 1
                @pl.when(nxt < _NCHUNK)
                def _():
                    _issue_in(nxt, prev)
        return 0

    lax.fori_loop(0, _NCHUNK // _NBUF, _outer, 0)
    _wait_scat(_NBUF - 1)

    plsc.subcore_barrier()
    pltpu.sync_copy(acc.at[pl.ds(sid * _SLICE, _SLICE)],
                    out_hbm.at[cid, pl.ds(sid * _SLICE, _SLICE)])


_sc_zbl = functools.partial(
    pl.kernel,
    out_type=jax.ShapeDtypeStruct((_NC, _NPAD), jnp.float32),
    mesh=plsc.VectorSubcoreMesh(core_axis_name="c", subcore_axis_name="s",
                                num_cores=_NC, num_subcores=_NS),
    scratch_types=[
        pltpu.VMEM((_N_NODES // 4,), jnp.int32),
        pltpu.VMEM((128,), jnp.float32),
        pltpu.VMEM((128,), jnp.float32),
        pltpu.VMEM((128,), jnp.float32),
        pltpu.VMEM((_PHI_N,), jnp.float32),
        pltpu.VMEM((_PHI_N,), jnp.float32),
        [pltpu.VMEM((_CHUNK,), jnp.float32) for _ in range(_NBUF)],
        [pltpu.VMEM((_CHUNK,), jnp.int32) for _ in range(_NBUF)],
        [pltpu.VMEM((_CHUNK,), jnp.int32) for _ in range(_NBUF)],
        [pltpu.VMEM((_CHUNK,), jnp.float32) for _ in range(2)],
        pltpu.VMEM_SHARED((_NPAD,), jnp.float32),
        [pltpu.SemaphoreType.DMA for _ in range(_NBUF)],
        [pltpu.SemaphoreType.DMA for _ in range(_NBUF)],
    ],
    compiler_params=pltpu.CompilerParams(needs_layout_passes=False),
)(_sc_body)


def _tc_add_body(p_ref, o_ref):
    o_ref[...] = p_ref[0, :] + p_ref[1, :]


_tc_add = pl.pallas_call(
    _tc_add_body,
    out_shape=jax.ShapeDtypeStruct((_NPAD,), jnp.float32),
)


def kernel(x, node_attrs, edge_index, atomic_numbers, node_attrs_index):
    del node_attrs
    xf = x.reshape(-1)
    s = edge_index[0]
    r = edge_index[1]
    e = node_attrs_index.astype(jnp.int32)
    elem = (e[0::4] | (e[1::4] << 8) | (e[2::4] << 16) | (e[3::4] << 24))

    zf = atomic_numbers.astype(jnp.float32)
    pz = jnp.power(zf, 0.3)
    a = (0.4543 * 0.529) / (pz[:, None] + pz[None, :])
    inva = (_PHI_SCALE / a).reshape(-1)
    cov = jnp.asarray(_COVALENT_RADII, jnp.float32)[atomic_numbers]
    invr = (1.0 / (cov[:, None] + cov[None, :])).reshape(-1)
    zz = ((0.5 * 14.3996) * (zf[:, None] * zf[None, :])).reshape(-1)
    pad = lambda t: jnp.pad(t, (0, 128 - t.shape[0]))

    partial = _sc_zbl(xf, s, r, elem, pad(inva), pad(invr), pad(zz),
                      jnp.asarray(_PHI_TAB), jnp.asarray(_DPHI_TAB))
    return _tc_add(partial)[:_N_NODES]


# speedup vs baseline: 1.2094x; 1.2094x over previous
import functools

import jax
import jax.numpy as jnp
import numpy as np
from jax import lax
from jax.experimental import pallas as pl
from jax.experimental.pallas import tpu as pltpu
from jax.experimental.pallas import tpu_sc as plsc

_COV_BASE = [0.2, 0.31, 0.28, 1.28, 0.96, 0.84, 0.76, 0.71, 0.66, 0.57, 0.58, 1.66, 1.41, 1.21, 1.11, 1.07, 1.05, 1.02, 1.06, 2.03, 1.76, 1.70, 1.60, 1.53, 1.39, 1.39, 1.32, 1.26, 1.24, 1.32, 1.22, 1.22, 1.20, 1.19, 1.20, 1.20, 1.16, 2.20, 1.95, 1.90, 1.75, 1.64, 1.54, 1.47, 1.46, 1.42, 1.39, 1.45, 1.44, 1.42, 1.39, 1.39, 1.38, 1.39, 1.40, 2.44, 2.15, 2.07, 2.04, 2.03, 2.01, 1.99, 1.98, 1.98, 1.96, 1.94, 1.92, 1.92, 1.89, 1.90, 1.87, 1.87, 1.75, 1.70, 1.62, 1.51, 1.44, 1.41, 1.36, 1.36, 1.32, 1.45, 1.46, 1.48, 1.40, 1.50, 1.50, 2.60, 2.21, 2.15, 2.06, 2.00, 1.96, 1.90, 1.87, 1.80, 1.69]
_COVALENT_RADII = np.array(_COV_BASE + [1.69] * (119 - len(_COV_BASE)), dtype=np.float32)

_N_NODES = 100000
_N_EDGES = 6400000
_NC = 2
_NS = 16
_NW = _NC * _NS
_EPW = _N_EDGES // _NW
_CHUNK = 2000
_NCHUNK = _EPW // _CHUNK
_NPAD = 100352
_SLICE = _NPAD // _NS
_L = 16


_NBUF = 4
_UNROLL = 5

_HK = 672
_HROW = _HK + 1
_HN = 100 * _HROW
_HN_PAD = 67304


def _sc_body(x_hbm, s_hbm, r_hbm, elem_hbm, h_hbm, zz_hbm,
             out_hbm, elem_v, h_v, zz_v,
             xbufs, sbufs, rbufs, vbufs, acc, in_sems,
             scat_sems):
    cid = lax.axis_index("c")
    sid = lax.axis_index("s")
    wid = cid * _NS + sid

    pltpu.sync_copy(elem_hbm, elem_v)
    pltpu.sync_copy(h_hbm, h_v)
    pltpu.sync_copy(zz_hbm, zz_v)

    def _zero(i, _):
        vbufs[0][pl.ds(i * _L, _L)] = jnp.zeros((_L,), jnp.float32)
        return 0
    lax.fori_loop(0, _CHUNK // _L, _zero, 0)
    for j in range(4):
        pltpu.sync_copy(vbufs[0].at[pl.ds(0, 1568)],
                        acc.at[pl.ds(sid * _SLICE + j * 1568, 1568)])
    plsc.subcore_barrier()

    base_e = wid * _EPW

    def _issue_in(c, slot):
        off = base_e + c * _CHUNK
        pltpu.async_copy(x_hbm.at[pl.ds(off, _CHUNK)], xbufs[slot],
                         in_sems[slot])
        pltpu.async_copy(s_hbm.at[pl.ds(off, _CHUNK)], sbufs[slot],
                         in_sems[slot])
        pltpu.async_copy(r_hbm.at[pl.ds(off, _CHUNK)], rbufs[slot],
                         in_sems[slot])

    def _wait_in(slot):
        pltpu.make_async_copy(x_hbm.at[pl.ds(0, _CHUNK)], xbufs[slot],
                              in_sems[slot]).wait()
        pltpu.make_async_copy(s_hbm.at[pl.ds(0, _CHUNK)], sbufs[slot],
                              in_sems[slot]).wait()
        pltpu.make_async_copy(r_hbm.at[pl.ds(0, _CHUNK)], rbufs[slot],
                              in_sems[slot]).wait()

    def _wait_scat(slot):
        pltpu.make_async_copy(vbufs[slot % 2], acc.at[rbufs[slot]],
                              scat_sems[slot]).wait()

    for b in range(_NBUF - 1):
        _issue_in(jnp.int32(b), b)

    def _compute(slot):
        xbuf, sbuf, rbuf = xbufs[slot], sbufs[slot], rbufs[slot]
        vbuf = vbufs[slot % 2]

        @plsc.parallel_loop(0, _CHUNK // _L, unroll=_UNROLL)
        def _vreg(i):
            o = i * _L
            s = sbuf[pl.ds(o, _L)]
            r = rbuf[pl.ds(o, _L)]
            wu = plsc.load_gather(elem_v, [s >> 2])
            wv = plsc.load_gather(elem_v, [r >> 2])
            eu = (wu >> ((s & 3) << 3)) & 0xFF
            ev = (wv >> ((r & 3) << 3)) & 0xFF
            pair = eu * 10 + ev
            zz = plsc.load_gather(zz_v, [pair])
            xv = xbuf[pl.ds(o, _L)]
            xs = xv * float(_HK)
            xi = xs.astype(jnp.int32)
            frac = xs - xi.astype(jnp.float32)
            idx = pair * _HROW + xi
            h0 = plsc.load_gather(h_v, [idx])
            h1 = plsc.load_gather(h_v, [idx + 1])
            hval = h0 + frac * (h1 - h0)
            vbuf[pl.ds(o, _L)] = zz * hval / xv

    def _outer(i, _):
        for b in range(_NBUF):
            _wait_in(b)
            _compute(b)
            pltpu.async_copy(vbufs[b % 2], acc.at[rbufs[b]], scat_sems[b],
                             add=True)
            prev = (b + _NBUF - 1) % _NBUF
            if b == 0:
                @pl.when(i > 0)
                def _():
                    _wait_scat(prev)
                    _issue_in(_NBUF * i + b + _NBUF - 1, prev)
                @pl.when(i == 0)
                def _():
                    _issue_in(_NBUF * i + b + _NBUF - 1, prev)
            else:
                _wait_scat(prev)
                nxt = _NBUF * i + b + _NBUF - 1
                @pl.when(nxt < _NCHUNK)
                def _():
                    _issue_in(nxt, prev)
        return 0

    lax.fori_loop(0, _NCHUNK // _NBUF, _outer, 0)
    _wait_scat(_NBUF - 1)

    plsc.subcore_barrier()
    pltpu.sync_copy(acc.at[pl.ds(sid * _SLICE, _SLICE)],
                    out_hbm.at[cid, pl.ds(sid * _SLICE, _SLICE)])


_sc_zbl = functools.partial(
    pl.kernel,
    out_type=jax.ShapeDtypeStruct((_NC, _NPAD), jnp.float32),
    mesh=plsc.VectorSubcoreMesh(core_axis_name="c", subcore_axis_name="s",
                                num_cores=_NC, num_subcores=_NS),
    scratch_types=[
        pltpu.VMEM((_N_NODES // 4,), jnp.int32),
        pltpu.VMEM((_HN_PAD,), jnp.float32),
        pltpu.VMEM((128,), jnp.float32),
        [pltpu.VMEM((_CHUNK,), jnp.float32) for _ in range(_NBUF)],
        [pltpu.VMEM((_CHUNK,), jnp.int32) for _ in range(_NBUF)],
        [pltpu.VMEM((_CHUNK,), jnp.int32) for _ in range(_NBUF)],
        [pltpu.VMEM((_CHUNK,), jnp.float32) for _ in range(2)],
        pltpu.VMEM_SHARED((_NPAD,), jnp.float32),
        [pltpu.SemaphoreType.DMA for _ in range(_NBUF)],
        [pltpu.SemaphoreType.DMA for _ in range(_NBUF)],
    ],
    compiler_params=pltpu.CompilerParams(needs_layout_passes=False),
)(_sc_body)


def _tc_add_body(p_ref, o_ref):
    o_ref[...] = p_ref[0, :] + p_ref[1, :]


_tc_add = pl.pallas_call(
    _tc_add_body,
    out_shape=jax.ShapeDtypeStruct((_NPAD,), jnp.float32),
)


def kernel(x, node_attrs, edge_index, atomic_numbers, node_attrs_index):
    del node_attrs
    xf = x.reshape(-1)
    s = edge_index[0]
    r = edge_index[1]
    e = node_attrs_index.astype(jnp.int32)
    elem = (e[0::4] | (e[1::4] << 8) | (e[2::4] << 16) | (e[3::4] << 24))

    zf = atomic_numbers.astype(jnp.float32)
    pz = jnp.power(zf, 0.3)
    a = (0.4543 * 0.529) / (pz[:, None] + pz[None, :])
    inva = (1.0 / a).reshape(-1)
    cov = jnp.asarray(_COVALENT_RADII, jnp.float32)[atomic_numbers]
    invr = (1.0 / (cov[:, None] + cov[None, :])).reshape(-1)
    zz = ((0.5 * 14.3996) * (zf[:, None] * zf[None, :])).reshape(-1)

    xg = (jnp.arange(_HROW, dtype=jnp.float32) / float(_HK))
    u = inva[:, None] * xg[None, :]
    t = invr[:, None] * xg[None, :]
    phi = (0.1818 * jnp.exp(-3.2 * u)
           + 0.5099 * jnp.exp(-0.9423 * u)
           + 0.2802 * jnp.exp(-0.4028 * u)
           + 0.02817 * jnp.exp(-0.2016 * u))
    t6 = t ** 6
    env = (1.0 - 28.0 * t6 + 48.0 * t6 * t - 21.0 * t6 * t * t) * (t < 1.0)
    h = (phi * env).reshape(-1)
    h = jnp.pad(h, (0, _HN_PAD - _HN))
    pad = lambda v: jnp.pad(v, (0, 128 - v.shape[0]))

    partial = _sc_zbl(xf, s, r, elem, h, pad(zz))
    return _tc_add(partial)[:_N_NODES]
